# baseline (speedup 1.0000x reference)
import numpy as np
import jax
import jax.numpy as jnp
from jax.experimental import pallas as pl
from jax.experimental.pallas import tpu as pltpu

B = 128
DEPTH = 9
NODES = 2 ** (DEPTH + 1) - 1
LEAVES = 2 ** DEPTH
INNER = NODES - LEAVES
N_FEAT = 4
N_HID = 64
JF = 4
W = JF * N_HID
G = 8
JPP = JF * G
NPROG = B // JPP

_bf = jnp.bfloat16
_f32 = jnp.float32


def _body(c_ref, wu_ref, bd3_ref, bu_ref, bh_ref, o_ref, *scr):
    c = c_ref[:, 0, 0, :]
    vs = []
    for g in range(G):
        u = jnp.tanh(jnp.dot(c, wu_ref[g], preferred_element_type=_f32)
                     + bu_ref[...])
        v = (jnp.dot(u[:INNER].astype(_bf), bd3_ref[2],
                     preferred_element_type=_f32)
             + bh_ref[...])
        leaves = u[INNER:]
        scr[g][0, :, :] = leaves[:, :128]
        scr[g][1, :, :] = leaves[:, 128:]
        vs.append(v)
    new = [None] * G
    for d in range(DEPTH - 1, -1, -1):
        n = 2 ** d
        for g in range(G):
            h_l = jnp.concatenate(
                [scr[g][0, pl.ds(0, n, 2), :], scr[g][1, pl.ds(0, n, 2), :]],
                axis=1).astype(_bf)
            h_r = jnp.concatenate(
                [scr[g][0, pl.ds(1, n, 2), :], scr[g][1, pl.ds(1, n, 2), :]],
                axis=1).astype(_bf)
            new[g] = jnp.tanh(
                jnp.dot(h_l, bd3_ref[0], preferred_element_type=_f32)
                + jnp.dot(h_r, bd3_ref[1], preferred_element_type=_f32)
                + vs[g][n - 1:2 * n - 1])
        if d > 0:
            for g in range(G):
                scr[g][0, pl.ds(0, n), :] = new[g][:, :128]
                scr[g][1, pl.ds(0, n), :] = new[g][:, 128:]
    for g in range(G):
        o_ref[g] = new[g]


_S = np.zeros((G, JPP * N_FEAT, JF * N_FEAT), np.float32)
for _g in range(G):
    for _t in range(JF * N_FEAT):
        _S[_g, JF * N_FEAT * _g + _t, _t] = 1.0


def _bd4(x):
    a, b = x.shape[-2:]
    eye = jnp.eye(JF, dtype=x.dtype)
    t = (eye[..., :, None, :, None] * x[..., None, :, None, :])
    return t.reshape(*x.shape[:-2], JF * a, JF * b)


def kernel(content, Wu, bu, Wh, bh):
    c_w = (content.reshape(B, NODES, N_FEAT).transpose(1, 0, 2)
           .reshape(NODES, NPROG, 1, JPP * N_FEAT).astype(_bf))
    Wu_bd = _bd4(Wu.T)
    Wu_all = jnp.einsum('gkt,th->gkh', jnp.asarray(_S), Wu_bd).astype(_bf)
    BD3 = _bd4(Wh.T.reshape(3, N_HID, N_HID)).astype(_bf)
    bu_t = jnp.tile(bu, JF).reshape(1, W)
    bh_t = jnp.tile(bh, JF).reshape(1, W)

    out = pl.pallas_call(
        _body,
        grid=(NPROG,),
        in_specs=[
            pl.BlockSpec((NODES, 1, 1, JPP * N_FEAT), lambda i: (0, i, 0, 0)),
            pl.BlockSpec((G, JPP * N_FEAT, W), lambda i: (0, 0, 0)),
            pl.BlockSpec((3, W, W), lambda i: (0, 0, 0)),
            pl.BlockSpec((1, W), lambda i: (0, 0)),
            pl.BlockSpec((1, W), lambda i: (0, 0)),
        ],
        out_specs=pl.BlockSpec((G, 1, W), lambda i: (i, 0, 0)),
        out_shape=jax.ShapeDtypeStruct((NPROG * G, 1, W), jnp.float32),
        scratch_shapes=[pltpu.VMEM((2, LEAVES, 128), jnp.float32)
                        for _ in range(G)],
    )(c_w, Wu_all, BD3, bu_t, bh_t)
    return out.reshape(B, N_HID)
